# Initial kernel scaffold; baseline (speedup 1.0000x reference)
import jax
import jax.numpy as jnp
from jax.experimental import pallas as pl


def kernel(feat, Wg, bg, Wf, bf, segment_ids):
    raise NotImplementedError("write your pallas kernel here")


# speedup vs baseline: 13.6754x; 13.6754x over previous
import functools

import jax
import jax.numpy as jnp
from jax.experimental import pallas as pl
from jax.experimental.pallas import tpu as pltpu

N_NODES = 50000
D_FEAT = 512
NUM_SEGMENTS = 256
BN = 2000
NB = N_NODES // BN
NEG = -1e30


def _pool_kernel(feat_ref, seg_ref, wg_ref, wf_ref, bf_ref, out_ref,
                 m_run, d_run, acc):
    k = pl.program_id(0)

    @pl.when(k == 0)
    def _init():
        m_run[...] = jnp.full((1, NUM_SEGMENTS), NEG, jnp.float32)
        d_run[...] = jnp.zeros((1, NUM_SEGMENTS), jnp.float32)
        acc[...] = jnp.zeros((D_FEAT, NUM_SEGMENTS), jnp.float32)

    feat = feat_ref[...]
    seg = seg_ref[0]
    g = jax.lax.dot_general(feat, wg_ref[...],
                            (((1,), (0,)), ((), ())),
                            preferred_element_type=jnp.float32)
    cols = jax.lax.broadcasted_iota(jnp.int32, (BN, NUM_SEGMENTS), 1)
    member = seg == cols
    g_masked = jnp.where(member, g, NEG)
    m_blk = jnp.max(g_masked, axis=0, keepdims=True)
    m_new = jnp.maximum(m_run[...], m_blk)
    scale = jnp.exp(jnp.maximum(m_run[...] - m_new, NEG))
    m_row = jnp.max(jnp.where(member, m_new, NEG), axis=1, keepdims=True)
    e = jnp.exp(g - m_row)
    w = jnp.where(member, e, 0.0)
    d_run[...] = d_run[...] * scale + jnp.sum(w, axis=0, keepdims=True)
    accT = jax.lax.dot_general(feat, w, (((0,), (0,)), ((), ())),
                               preferred_element_type=jnp.float32)
    acc[...] = acc[...] * scale + accT
    m_run[...] = m_new

    @pl.when(k == NB - 1)
    def _finish():
        d = d_run[...]
        inv = jnp.where(d > 0.0, 1.0 / d, 0.0)
        pooledT = acc[...] * inv
        out = jax.lax.dot_general(pooledT, wf_ref[...],
                                  (((0,), (0,)), ((), ())),
                                  preferred_element_type=jnp.float32)
        r = jax.lax.broadcasted_iota(jnp.int32, (NUM_SEGMENTS, NUM_SEGMENTS), 0)
        c = jax.lax.broadcasted_iota(jnp.int32, (NUM_SEGMENTS, NUM_SEGMENTS), 1)
        d_col = jnp.sum(jnp.where(r == c, jnp.broadcast_to(d, (NUM_SEGMENTS, NUM_SEGMENTS)), 0.0),
                        axis=1, keepdims=True)
        out_ref[...] = out + jnp.where(d_col > 0.0, bf_ref[...], 0.0)


@jax.jit
def kernel(feat, Wg, bg, Wf, bf, segment_ids):
    del bg
    seg3 = segment_ids.astype(jnp.int32).reshape(NB, BN, 1)
    bf2 = bf.reshape(1, D_FEAT)
    grid = (NB,)
    out = pl.pallas_call(
        _pool_kernel,
        grid=grid,
        in_specs=[
            pl.BlockSpec((BN, D_FEAT), lambda k: (k, 0)),
            pl.BlockSpec((1, BN, 1), lambda k: (k, 0, 0)),
            pl.BlockSpec((D_FEAT, 1), lambda k: (0, 0)),
            pl.BlockSpec((D_FEAT, D_FEAT), lambda k: (0, 0)),
            pl.BlockSpec((1, D_FEAT), lambda k: (0, 0)),
        ],
        out_specs=pl.BlockSpec((NUM_SEGMENTS, D_FEAT), lambda k: (0, 0)),
        out_shape=jax.ShapeDtypeStruct((NUM_SEGMENTS, D_FEAT), jnp.float32),
        scratch_shapes=[
            pltpu.VMEM((1, NUM_SEGMENTS), jnp.float32),
            pltpu.VMEM((1, NUM_SEGMENTS), jnp.float32),
            pltpu.VMEM((D_FEAT, NUM_SEGMENTS), jnp.float32),
        ],
    )(feat, seg3, Wg, Wf, bf2)
    return out
